# baseline (speedup 1.0000x reference)
import functools

import jax
import jax.numpy as jnp
from jax import lax
from jax.experimental import pallas as pl
from jax.experimental.pallas import tpu as pltpu
import jax.experimental.pallas.tpu_sc as plsc

_NC = 2
_NS = 16
_NW = _NC * _NS
_C = 128


def _scale_body(h_ref, norm_ref, o_ref):
    o_ref[...] = h_ref[...] * norm_ref[...]


_U = 3


def _make_sc_body(w_per, rps, n_pad):
    def body(norm_h_hbm, src_hbm, dst_hbm, z_hbm, out_hbm,
             src0, src1, src2, dst0, dst1, dst2, rows0, rows1, rows2, acc_sh,
             is0, is1, is2, id0, id1, id2, gs0, gs1, gs2, ss0, ss1, ss2):
        c = lax.axis_index("c")
        s = lax.axis_index("s")
        wid = s * _NC + c
        row0 = pl.multiple_of(s * rps, 8)
        srcs, dsts, rows = (src0, src1, src2), (dst0, dst1, dst2), (rows0, rows1, rows2)
        isems, idems, gsems, ssems = (is0, is1, is2), (id0, id1, id2), (gs0, gs1, gs2), (ss0, ss1, ss2)
        pltpu.sync_copy(z_hbm, acc_sh.at[pl.ds(row0, rps)])
        plsc.subcore_barrier()

        def step(i, carry):
            base = wid * w_per + i * (_U * _C)
            ia, ib = [], []
            for u in range(_U):
                off = pl.multiple_of(base + u * _C, 8)
                ia.append(pltpu.async_copy(src_hbm.at[pl.ds(off, _C)], srcs[u], isems[u]))
                ib.append(pltpu.async_copy(dst_hbm.at[pl.ds(off, _C)], dsts[u], idems[u]))
            g = []
            for u in range(_U):
                ia[u].wait()
                g.append(pltpu.async_copy(norm_h_hbm.at[srcs[u]], rows[u], gsems[u]))
            sd = []
            for u in range(_U):
                g[u].wait()
                ib[u].wait()
                sd.append(pltpu.async_copy(rows[u], acc_sh.at[dsts[u]], ssems[u], add=True))
            for u in range(_U):
                sd[u].wait()
            return carry

        lax.fori_loop(0, w_per // (_U * _C), step, 0)
        plsc.subcore_barrier()
        out0 = pl.multiple_of(c * n_pad + s * rps, 8)
        pltpu.sync_copy(acc_sh.at[pl.ds(row0, rps)], out_hbm.at[pl.ds(out0, rps)])

    return body


def _mm_body(h_ref, p_ref, norm_ref, w_ref, w2_ref, o_ref):
    nrm = norm_ref[...]
    agg = (p_ref[0] + p_ref[1]) * nrm
    x = jnp.concatenate([h_ref[...], agg], axis=1)
    y = jnp.dot(x, w_ref[...], preferred_element_type=jnp.float32)
    y = jnp.maximum(y, 0.0)
    ss = jnp.sum(y * y, axis=1, keepdims=True)
    y = y * lax.rsqrt(jnp.maximum(ss, 1e-12))
    o_ref[...] = jnp.maximum(
        jnp.dot(y, w2_ref[...], preferred_element_type=jnp.float32), 0.0)


def kernel(h, edge_index, norm, weight, weight2):
    n, d = h.shape
    e = edge_index.shape[1]
    d_out = weight2.shape[1]

    src = edge_index[0].astype(jnp.int32)
    dst = edge_index[1].astype(jnp.int32)

    w_per = -(-e // (_NW * _C * _U)) * (_C * _U)
    e_pad = w_per * _NW
    n_pad = -(-n // _C) * _C
    rps = n_pad // _NS
    if e_pad > e:
        src = jnp.concatenate([src, jnp.zeros((e_pad - e,), jnp.int32)])
        dst = jnp.concatenate([dst, jnp.full((e_pad - e,), n, jnp.int32)])

    bn = 1000 if n % 1000 == 0 else n

    norm_h = pl.pallas_call(
        _scale_body,
        out_shape=jax.ShapeDtypeStruct((n, d), jnp.float32),
        grid=(n // bn,),
        in_specs=[pl.BlockSpec((bn, d), lambda i: (i, 0)),
                  pl.BlockSpec((bn, 1), lambda i: (i, 0))],
        out_specs=pl.BlockSpec((bn, d), lambda i: (i, 0)),
    )(h, norm)

    z = jnp.zeros((rps, d), jnp.float32)
    mesh = plsc.VectorSubcoreMesh(core_axis_name="c", subcore_axis_name="s")
    partial = pl.kernel(
        _make_sc_body(w_per, rps, n_pad),
        out_type=jax.ShapeDtypeStruct((_NC * n_pad, d), jnp.float32),
        mesh=mesh,
        scratch_types=(
            [pltpu.VMEM((_C,), jnp.int32)] * (2 * _U)
            + [pltpu.VMEM((_C, d), jnp.float32)] * _U
            + [pltpu.VMEM_SHARED((n_pad, d), jnp.float32)]
            + [pltpu.SemaphoreType.DMA] * (4 * _U)
        ),
    )(norm_h, src, dst, z)
    p = partial.reshape(_NC, n_pad, d)

    out = pl.pallas_call(
        _mm_body,
        out_shape=jax.ShapeDtypeStruct((n, d_out), jnp.float32),
        grid=(n // bn,),
        in_specs=[
            pl.BlockSpec((bn, d), lambda i: (i, 0)),
            pl.BlockSpec((_NC, bn, d), lambda i: (0, i, 0)),
            pl.BlockSpec((bn, 1), lambda i: (i, 0)),
            pl.BlockSpec(weight.shape, lambda i: (0, 0)),
            pl.BlockSpec(weight2.shape, lambda i: (0, 0)),
        ],
        out_specs=pl.BlockSpec((bn, d_out), lambda i: (i, 0)),
    )(h, p, norm, weight, weight2)
    return out


# speedup vs baseline: 1.2370x; 1.2370x over previous
import functools

import jax
import jax.numpy as jnp
from jax import lax
from jax.experimental import pallas as pl
from jax.experimental.pallas import tpu as pltpu
import jax.experimental.pallas.tpu_sc as plsc

_NC = 2
_NS = 16
_NW = _NC * _NS
_C = 256


def _scale_body(h_ref, norm_ref, o_ref):
    o_ref[...] = h_ref[...] * norm_ref[...]


def _make_sc_body(w_per, rps, n_pad):
    def body(norm_h_hbm, src_hbm, dst_hbm, z_hbm, out_hbm,
             src_v, dst_v, rows_v, acc_sh, sem):
        c = lax.axis_index("c")
        s = lax.axis_index("s")
        wid = s * _NC + c
        row0 = pl.multiple_of(s * rps, 8)
        pltpu.sync_copy(z_hbm, acc_sh.at[pl.ds(row0, rps)])
        plsc.subcore_barrier()

        def step(i, carry):
            off = pl.multiple_of(wid * w_per + i * _C, 8)
            pltpu.sync_copy(src_hbm.at[pl.ds(off, _C)], src_v)
            pltpu.sync_copy(dst_hbm.at[pl.ds(off, _C)], dst_v)
            pltpu.async_copy(norm_h_hbm.at[src_v], rows_v, sem).wait()
            pltpu.sync_copy(rows_v, acc_sh.at[dst_v], add=True)
            return carry

        lax.fori_loop(0, w_per // _C, step, 0)
        plsc.subcore_barrier()
        out0 = pl.multiple_of(c * n_pad + s * rps, 8)
        pltpu.sync_copy(acc_sh.at[pl.ds(row0, rps)], out_hbm.at[pl.ds(out0, rps)])

    return body


def _mm_body(h_ref, p_ref, norm_ref, w_ref, w2_ref, o_ref):
    nrm = norm_ref[...]
    agg = (p_ref[0] + p_ref[1]) * nrm
    x = jnp.concatenate([h_ref[...], agg], axis=1)
    y = jnp.dot(x, w_ref[...], preferred_element_type=jnp.float32)
    y = jnp.maximum(y, 0.0)
    ss = jnp.sum(y * y, axis=1, keepdims=True)
    y = y * lax.rsqrt(jnp.maximum(ss, 1e-12))
    o_ref[...] = jnp.maximum(
        jnp.dot(y, w2_ref[...], preferred_element_type=jnp.float32), 0.0)


def kernel(h, edge_index, norm, weight, weight2):
    n, d = h.shape
    e = edge_index.shape[1]
    d_out = weight2.shape[1]

    src = edge_index[0].astype(jnp.int32)
    dst = edge_index[1].astype(jnp.int32)

    w_per = -(-e // (_NW * _C)) * _C
    e_pad = w_per * _NW
    n_pad = -(-n // _C) * _C
    rps = n_pad // _NS
    if e_pad > e:
        src = jnp.concatenate([src, jnp.zeros((e_pad - e,), jnp.int32)])
        dst = jnp.concatenate([dst, jnp.full((e_pad - e,), n, jnp.int32)])

    bn = 1000 if n % 1000 == 0 else n

    norm_h = pl.pallas_call(
        _scale_body,
        out_shape=jax.ShapeDtypeStruct((n, d), jnp.float32),
        grid=(n // bn,),
        in_specs=[pl.BlockSpec((bn, d), lambda i: (i, 0)),
                  pl.BlockSpec((bn, 1), lambda i: (i, 0))],
        out_specs=pl.BlockSpec((bn, d), lambda i: (i, 0)),
    )(h, norm)

    z = jnp.zeros((rps, d), jnp.float32)
    mesh = plsc.VectorSubcoreMesh(core_axis_name="c", subcore_axis_name="s")
    partial = pl.kernel(
        _make_sc_body(w_per, rps, n_pad),
        out_type=jax.ShapeDtypeStruct((_NC * n_pad, d), jnp.float32),
        mesh=mesh,
        scratch_types=[
            pltpu.VMEM((_C,), jnp.int32),
            pltpu.VMEM((_C,), jnp.int32),
            pltpu.VMEM((_C, d), jnp.float32),
            pltpu.VMEM_SHARED((n_pad, d), jnp.float32),
            pltpu.SemaphoreType.DMA,
        ],
    )(norm_h, src, dst, z)
    p = partial.reshape(_NC, n_pad, d)

    out = pl.pallas_call(
        _mm_body,
        out_shape=jax.ShapeDtypeStruct((n, d_out), jnp.float32),
        grid=(n // bn,),
        in_specs=[
            pl.BlockSpec((bn, d), lambda i: (i, 0)),
            pl.BlockSpec((_NC, bn, d), lambda i: (0, i, 0)),
            pl.BlockSpec((bn, 1), lambda i: (i, 0)),
            pl.BlockSpec(weight.shape, lambda i: (0, 0)),
            pl.BlockSpec(weight2.shape, lambda i: (0, 0)),
        ],
        out_specs=pl.BlockSpec((bn, d_out), lambda i: (i, 0)),
    )(h, p, norm, weight, weight2)
    return out


# speedup vs baseline: 1.2663x; 1.0237x over previous
import jax
import jax.numpy as jnp
from jax import lax
from jax.experimental import pallas as pl
from jax.experimental.pallas import tpu as pltpu
import jax.experimental.pallas.tpu_sc as plsc

_NC = 2
_NS = 16
_NW = _NC * _NS
_C = 128


def _scale_body(h_ref, norm_ref, o_ref):
    o_ref[...] = h_ref[...] * norm_ref[...]


def _make_sc_body(w_per, rps, n_pad):
    n_steps = w_per // (2 * _C)

    def body(norm_h_hbm, src_hbm, dst_hbm, z_hbm, out_hbm,
             src_a, dst_a, src_b, dst_b, rows_v, acc_sh,
             gsem, sa, da, sb, db):
        c = lax.axis_index("c")
        s = lax.axis_index("s")
        wid = s * _NC + c
        row0 = pl.multiple_of(s * rps, 8)
        ebase = pl.multiple_of(wid * w_per, 8)
        pltpu.sync_copy(z_hbm, acc_sh.at[pl.ds(row0, rps)])
        pltpu.sync_copy(src_hbm.at[pl.ds(ebase, _C)], src_a)
        pltpu.sync_copy(dst_hbm.at[pl.ds(ebase, _C)], dst_a)
        plsc.subcore_barrier()

        def step(i, carry):
            off = pl.multiple_of(ebase + i * (2 * _C), 8)
            pb = (pltpu.async_copy(src_hbm.at[pl.ds(off + _C, _C)], src_b, sb),
                  pltpu.async_copy(dst_hbm.at[pl.ds(off + _C, _C)], dst_b, db))
            pltpu.async_copy(norm_h_hbm.at[src_a], rows_v, gsem).wait()
            pltpu.sync_copy(rows_v, acc_sh.at[dst_a], add=True)
            pa = (pltpu.async_copy(src_hbm.at[pl.ds(off + 2 * _C, _C)], src_a, sa),
                  pltpu.async_copy(dst_hbm.at[pl.ds(off + 2 * _C, _C)], dst_a, da))
            pb[0].wait()
            pb[1].wait()
            pltpu.async_copy(norm_h_hbm.at[src_b], rows_v, gsem).wait()
            pltpu.sync_copy(rows_v, acc_sh.at[dst_b], add=True)
            pa[0].wait()
            pa[1].wait()
            return carry

        lax.fori_loop(0, n_steps, step, 0)
        plsc.subcore_barrier()
        out0 = pl.multiple_of(c * n_pad + s * rps, 8)
        pltpu.sync_copy(acc_sh.at[pl.ds(row0, rps)], out_hbm.at[pl.ds(out0, rps)])

    return body


def _mm_body(h_ref, p_ref, norm_ref, w_ref, w2_ref, o_ref):
    nrm = norm_ref[...]
    agg = (p_ref[0] + p_ref[1]) * nrm
    x = jnp.concatenate([h_ref[...], agg], axis=1)
    y = jnp.dot(x, w_ref[...], preferred_element_type=jnp.float32)
    y = jnp.maximum(y, 0.0)
    ss = jnp.sum(y * y, axis=1, keepdims=True)
    y = y * lax.rsqrt(jnp.maximum(ss, 1e-12))
    o_ref[...] = jnp.maximum(
        jnp.dot(y, w2_ref[...], preferred_element_type=jnp.float32), 0.0)


def kernel(h, edge_index, norm, weight, weight2):
    n, d = h.shape
    e = edge_index.shape[1]
    d_out = weight2.shape[1]

    src = edge_index[0].astype(jnp.int32)
    dst = edge_index[1].astype(jnp.int32)

    w_per = -(-e // (_NW * 2 * _C)) * (2 * _C)
    e_pad = w_per * _NW + _C
    n_pad = -(-n // (_NS * 8)) * (_NS * 8)
    rps = n_pad // _NS
    src = jnp.concatenate([src, jnp.zeros((e_pad - e,), jnp.int32)])
    dst = jnp.concatenate([dst, jnp.full((e_pad - e,), n, jnp.int32)])

    bn = 1000 if n % 1000 == 0 else n

    norm_h = pl.pallas_call(
        _scale_body,
        out_shape=jax.ShapeDtypeStruct((n, d), jnp.float32),
        grid=(n // bn,),
        in_specs=[pl.BlockSpec((bn, d), lambda i: (i, 0)),
                  pl.BlockSpec((bn, 1), lambda i: (i, 0))],
        out_specs=pl.BlockSpec((bn, d), lambda i: (i, 0)),
    )(h, norm)

    z = jnp.zeros((rps, d), jnp.float32)
    mesh = plsc.VectorSubcoreMesh(core_axis_name="c", subcore_axis_name="s")
    partial = pl.kernel(
        _make_sc_body(w_per, rps, n_pad),
        out_type=jax.ShapeDtypeStruct((_NC * n_pad, d), jnp.float32),
        mesh=mesh,
        scratch_types=[
            pltpu.VMEM((_C,), jnp.int32),
            pltpu.VMEM((_C,), jnp.int32),
            pltpu.VMEM((_C,), jnp.int32),
            pltpu.VMEM((_C,), jnp.int32),
            pltpu.VMEM((_C, d), jnp.float32),
            pltpu.VMEM_SHARED((n_pad, d), jnp.float32),
            pltpu.SemaphoreType.DMA,
            pltpu.SemaphoreType.DMA,
            pltpu.SemaphoreType.DMA,
            pltpu.SemaphoreType.DMA,
            pltpu.SemaphoreType.DMA,
        ],
    )(norm_h, src, dst, z)
    p = partial.reshape(_NC, n_pad, d)

    out = pl.pallas_call(
        _mm_body,
        out_shape=jax.ShapeDtypeStruct((n, d_out), jnp.float32),
        grid=(n // bn,),
        in_specs=[
            pl.BlockSpec((bn, d), lambda i: (i, 0)),
            pl.BlockSpec((_NC, bn, d), lambda i: (0, i, 0)),
            pl.BlockSpec((bn, 1), lambda i: (i, 0)),
            pl.BlockSpec(weight.shape, lambda i: (0, 0)),
            pl.BlockSpec(weight2.shape, lambda i: (0, 0)),
        ],
        out_specs=pl.BlockSpec((bn, d_out), lambda i: (i, 0)),
    )(h, p, norm, weight, weight2)
    return out


# speedup vs baseline: 1.8758x; 1.4814x over previous
import jax
import jax.numpy as jnp
from jax import lax
from jax.experimental import pallas as pl
from jax.experimental.pallas import tpu as pltpu
import jax.experimental.pallas.tpu_sc as plsc

_NC = 2
_NS = 16
_NW = _NC * _NS
_C = 128
_SPLIT0 = 61


def _scale_body(h_ref, norm_ref, o_ref):
    o_ref[...] = h_ref[...] * norm_ref[...]


def _make_sc_body(w_pair, rps, n_pad):
    n_pair = w_pair // _C

    def body(norm_h_hbm, src_hbm, dst_hbm, z_hbm, out_hbm,
             src_v, dst_v, rows_v, acc_sh, sem):
        c = lax.axis_index("c")
        s = lax.axis_index("s")
        row0 = pl.multiple_of(s * rps, 8)
        pltpu.sync_copy(z_hbm, acc_sh.at[pl.ds(row0, rps)])
        plsc.subcore_barrier()

        ebase = pl.multiple_of(s * w_pair + c * (_SPLIT0 * _C), 8)
        nc = lax.select(c == 0, _SPLIT0, n_pair - _SPLIT0)

        def step(i, carry):
            off = pl.multiple_of(ebase + i * _C, 8)
            pltpu.sync_copy(src_hbm.at[pl.ds(off, _C)], src_v)
            pltpu.sync_copy(dst_hbm.at[pl.ds(off, _C)], dst_v)
            pltpu.async_copy(norm_h_hbm.at[src_v], rows_v, sem).wait()
            pltpu.sync_copy(rows_v, acc_sh.at[dst_v], add=True)
            return carry

        lax.fori_loop(0, nc, step, 0)
        plsc.subcore_barrier()
        out0 = pl.multiple_of(c * n_pad + s * rps, 8)
        pltpu.sync_copy(acc_sh.at[pl.ds(row0, rps)], out_hbm.at[pl.ds(out0, rps)])

    return body


def _mm_body(h_ref, p_ref, norm_ref, w_ref, w2_ref, o_ref):
    nrm = norm_ref[...]
    agg = (p_ref[0] + p_ref[1]) * nrm
    x = jnp.concatenate([h_ref[...], agg], axis=1)
    y = jnp.dot(x, w_ref[...], preferred_element_type=jnp.float32)
    y = jnp.maximum(y, 0.0)
    ss = jnp.sum(y * y, axis=1, keepdims=True)
    y = y * lax.rsqrt(jnp.maximum(ss, 1e-12))
    o_ref[...] = jnp.maximum(
        jnp.dot(y, w2_ref[...], preferred_element_type=jnp.float32), 0.0)


def kernel(h, edge_index, norm, weight, weight2):
    n, d = h.shape
    e = edge_index.shape[1]
    d_out = weight2.shape[1]

    src = edge_index[0].astype(jnp.int32)
    dst = edge_index[1].astype(jnp.int32)

    w_pair = -(-e // (_NS * _C)) * _C
    e_pad = w_pair * _NS
    n_pad = -(-n // (_NS * 8)) * (_NS * 8)
    rps = n_pad // _NS
    if e_pad > e:
        src = jnp.concatenate([src, jnp.zeros((e_pad - e,), jnp.int32)])
        dst = jnp.concatenate([dst, jnp.full((e_pad - e,), n, jnp.int32)])

    bn = 1000 if n % 1000 == 0 else n

    norm_h = pl.pallas_call(
        _scale_body,
        out_shape=jax.ShapeDtypeStruct((n, d), jnp.float32),
        grid=(n // bn,),
        in_specs=[pl.BlockSpec((bn, d), lambda i: (i, 0)),
                  pl.BlockSpec((bn, 1), lambda i: (i, 0))],
        out_specs=pl.BlockSpec((bn, d), lambda i: (i, 0)),
    )(h, norm)

    z = jnp.zeros((rps, d), jnp.float32)
    mesh = plsc.VectorSubcoreMesh(core_axis_name="c", subcore_axis_name="s")
    partial = pl.kernel(
        _make_sc_body(w_pair, rps, n_pad),
        out_type=jax.ShapeDtypeStruct((_NC * n_pad, d), jnp.float32),
        mesh=mesh,
        scratch_types=[
            pltpu.VMEM((_C,), jnp.int32),
            pltpu.VMEM((_C,), jnp.int32),
            pltpu.VMEM((_C, d), jnp.float32),
            pltpu.VMEM_SHARED((n_pad, d), jnp.float32),
            pltpu.SemaphoreType.DMA,
        ],
    )(norm_h, src, dst, z)
    p = partial.reshape(_NC, n_pad, d)

    out = pl.pallas_call(
        _mm_body,
        out_shape=jax.ShapeDtypeStruct((n, d_out), jnp.float32),
        grid=(n // bn,),
        in_specs=[
            pl.BlockSpec((bn, d), lambda i: (i, 0)),
            pl.BlockSpec((_NC, bn, d), lambda i: (0, i, 0)),
            pl.BlockSpec((bn, 1), lambda i: (i, 0)),
            pl.BlockSpec(weight.shape, lambda i: (0, 0)),
            pl.BlockSpec(weight2.shape, lambda i: (0, 0)),
        ],
        out_specs=pl.BlockSpec((bn, d_out), lambda i: (i, 0)),
    )(h, p, norm, weight, weight2)
    return out
